# Initial kernel scaffold; baseline (speedup 1.0000x reference)
import jax
import jax.numpy as jnp
from jax.experimental import pallas as pl


def kernel(user_ids, item_ids, user_emb, item_emb, global_bias):
    raise NotImplementedError("write your pallas kernel here")


# speedup vs baseline: 1.0547x; 1.0547x over previous
import functools

import jax
import jax.numpy as jnp
from jax import lax
from jax.experimental import pallas as pl
from jax.experimental.pallas import tpu as pltpu
from jax.experimental.pallas import tpu_sc as plsc

EMB = 128
LANES = 16
NCORES = 2
NSUB = 16
NW = NCORES * NSUB
BATCH = 16384
BPW = BATCH // NW
CHUNK = 128
NCHUNK = BPW // CHUNK
GROUPS = CHUNK // LANES

_mesh = plsc.VectorSubcoreMesh(core_axis_name="c", subcore_axis_name="s")


@functools.partial(
    pl.kernel,
    out_type=jax.ShapeDtypeStruct((BATCH,), jnp.float32),
    mesh=_mesh,
    compiler_params=pltpu.CompilerParams(needs_layout_passes=False),
    scratch_types=[
        pltpu.VMEM((BPW,), jnp.int32),
        pltpu.VMEM((BPW,), jnp.int32),
        pltpu.VMEM((CHUNK, EMB), jnp.float32),
        pltpu.VMEM((CHUNK, EMB), jnp.float32),
        pltpu.VMEM((BPW,), jnp.float32),
        pltpu.VMEM((LANES,), jnp.float32),
        pltpu.SemaphoreType.DMA,
    ],
)
def _bpr_sc(uids_hbm, iids_hbm, uemb_hbm, iemb_hbm, bias_hbm, out_hbm,
            uidx, iidx, urows, irows, outb, biasb, sem):
    wid = lax.axis_index("s") * NCORES + lax.axis_index("c")
    base = wid * BPW
    pltpu.sync_copy(bias_hbm, biasb)
    pltpu.sync_copy(uids_hbm.at[pl.ds(base, BPW)], uidx)
    pltpu.sync_copy(iids_hbm.at[pl.ds(base, BPW)], iidx)
    bias_vec = biasb[...]
    row_iota = lax.iota(jnp.int32, LANES)

    for c in range(NCHUNK):
        pltpu.async_copy(
            uemb_hbm.at[uidx.at[pl.ds(c * CHUNK, CHUNK)]], urows, sem
        ).wait()
        pltpu.async_copy(
            iemb_hbm.at[iidx.at[pl.ds(c * CHUNK, CHUNK)]], irows, sem
        ).wait()

        def group_body(g, _, c=c):
            ov = bias_vec
            for r in range(LANES):
                row = g * LANES + r
                a = urows[row, pl.ds(0, LANES)] * irows[row, pl.ds(0, LANES)]
                for j in range(1, EMB // LANES):
                    a = a + (urows[row, pl.ds(j * LANES, LANES)]
                             * irows[row, pl.ds(j * LANES, LANES)])
                ov = jnp.where(row_iota == r, ov + jnp.sum(a), ov)
            outb[pl.ds(c * CHUNK + g * LANES, LANES)] = ov
            return 0

        lax.fori_loop(0, GROUPS, group_body, 0)

    pltpu.sync_copy(outb, out_hbm.at[pl.ds(base, BPW)])


def kernel(user_ids, item_ids, user_emb, item_emb, global_bias):
    bias_vec = jnp.full((LANES,), global_bias, dtype=jnp.float32)
    return _bpr_sc(user_ids, item_ids, user_emb, item_emb, bias_vec)


# speedup vs baseline: 1.2706x; 1.2047x over previous
import functools

import jax
import jax.numpy as jnp
from jax import lax
from jax.experimental import pallas as pl
from jax.experimental.pallas import tpu as pltpu
from jax.experimental.pallas import tpu_sc as plsc

EMB = 128
LANES = 16
NCORES = 2
NSUB = 16
NW = NCORES * NSUB
BATCH = 16384
BPW = BATCH // NW
CHUNK = 128
NCHUNK = BPW // CHUNK
GROUPS = CHUNK // LANES

_mesh = plsc.VectorSubcoreMesh(core_axis_name="c", subcore_axis_name="s")


@functools.partial(
    pl.kernel,
    out_type=jax.ShapeDtypeStruct((BATCH,), jnp.float32),
    mesh=_mesh,
    compiler_params=pltpu.CompilerParams(needs_layout_passes=False),
    scratch_types=[
        pltpu.VMEM((BPW,), jnp.int32),
        pltpu.VMEM((BPW,), jnp.int32),
        pltpu.VMEM((CHUNK, EMB), jnp.float32),
        pltpu.VMEM((CHUNK, EMB), jnp.float32),
        pltpu.VMEM((CHUNK, EMB), jnp.float32),
        pltpu.VMEM((CHUNK, EMB), jnp.float32),
        pltpu.VMEM((BPW,), jnp.float32),
        pltpu.VMEM((LANES,), jnp.float32),
        pltpu.SemaphoreType.DMA,
        pltpu.SemaphoreType.DMA,
    ],
)
def _bpr_sc(uids_hbm, iids_hbm, uemb_hbm, iemb_hbm, bias_hbm, out_hbm,
            uidx, iidx, urows0, urows1, irows0, irows1, outb, biasb,
            sem0, sem1):
    wid = lax.axis_index("s") * NCORES + lax.axis_index("c")
    base = wid * BPW
    cp_b = pltpu.async_copy(bias_hbm, biasb, sem0)
    cp_u = pltpu.async_copy(uids_hbm.at[pl.ds(base, BPW)], uidx, sem0)
    cp_i = pltpu.async_copy(iids_hbm.at[pl.ds(base, BPW)], iidx, sem0)
    cp_b.wait()
    cp_u.wait()
    cp_i.wait()
    bias_vec = biasb[...]
    row_iota = lax.iota(jnp.int32, LANES)

    bufs = [(urows0, irows0, sem0), (urows1, irows1, sem1)]

    def issue(c):
        ub, ib, sem = bufs[c & 1]
        return (
            pltpu.async_copy(
                uemb_hbm.at[uidx.at[pl.ds(c * CHUNK, CHUNK)]], ub, sem),
            pltpu.async_copy(
                iemb_hbm.at[iidx.at[pl.ds(c * CHUNK, CHUNK)]], ib, sem),
        )

    pending = issue(0)
    for c in range(NCHUNK):
        nxt = issue(c + 1) if c + 1 < NCHUNK else ()
        for cp in pending:
            cp.wait()
        pending = nxt
        urows, irows, _ = bufs[c & 1]

        def group_body(g, _, c=c, urows=urows, irows=irows):
            ov = bias_vec
            for r in range(LANES):
                row = g * LANES + r
                a = urows[row, pl.ds(0, LANES)] * irows[row, pl.ds(0, LANES)]
                for j in range(1, EMB // LANES):
                    a = a + (urows[row, pl.ds(j * LANES, LANES)]
                             * irows[row, pl.ds(j * LANES, LANES)])
                ov = jnp.where(row_iota == r, ov + jnp.sum(a), ov)
            outb[pl.ds(c * CHUNK + g * LANES, LANES)] = ov
            return 0

        lax.fori_loop(0, GROUPS, group_body, 0)

    pltpu.sync_copy(outb, out_hbm.at[pl.ds(base, BPW)])


def kernel(user_ids, item_ids, user_emb, item_emb, global_bias):
    bias_vec = jnp.full((LANES,), global_bias, dtype=jnp.float32)
    return _bpr_sc(user_ids, item_ids, user_emb, item_emb, bias_vec)
